# baseline (speedup 1.0000x reference)
import jax
import jax.numpy as jnp
from jax import lax
from jax.experimental import pallas as pl
from jax.experimental.pallas import tpu as pltpu
from jax.experimental.pallas import tpu_sc as plsc

N_NODES = 10000
DIM = 128
N_EDGES = 320000

NC = 2
NS = 16
NW = NC * NS

EDGES_PER_W = N_EDGES // NW
CHUNK = 80
N_CHUNKS = EDGES_PER_W // CHUNK
BLK = 16


def _dot_chunk(zr_ref, zc_ref, out_ref):
    nblk = CHUNK // BLK
    rids = [lax.iota(jnp.int32, 16) + b * BLK for b in range(nblk)]

    def body(d, accs):
        dvec = jnp.full((16,), 0, jnp.int32) + d
        new = []
        for b in range(nblk):
            a = plsc.load_gather(zr_ref, [rids[b], dvec])
            c = plsc.load_gather(zc_ref, [rids[b], dvec])
            new.append(accs[b] + a * c)
        return tuple(new)

    accs = lax.fori_loop(
        0, DIM, body,
        tuple(jnp.zeros((16,), jnp.float32) for _ in range(nblk)),
        unroll=4,
    )
    for b in range(nblk):
        out_ref[pl.ds(b * BLK, BLK)] = 1.0 / (1.0 + jnp.exp(-accs[b]))


def _sc_body(z_hbm, row_hbm, col_hbm, out_hbm,
             idxr0, idxc0, idxr1, idxc1,
             zr0, zc0, zr1, zc1, outv0, outv1,
             sg0, sg1, si0, si1, so0, so1):
    idxr = (idxr0, idxr1)
    idxc = (idxc0, idxc1)
    zr = (zr0, zr1)
    zc = (zc0, zc1)
    outv = (outv0, outv1)
    sem_g = (sg0, sg1)
    sem_i = (si0, si1)
    sem_o = (so0, so1)

    wid = lax.axis_index("c") * NS + lax.axis_index("s")
    base = wid * EDGES_PER_W

    def fire_idx(c, p):
        ebase = base + c * CHUNK
        pltpu.async_copy(row_hbm.at[pl.ds(ebase, CHUNK)], idxr[p], sem_i[p])
        pltpu.async_copy(col_hbm.at[pl.ds(ebase, CHUNK)], idxc[p], sem_i[p])

    def wait_idx(c, p):
        ebase = base + c * CHUNK
        pltpu.make_async_copy(
            row_hbm.at[pl.ds(ebase, CHUNK)], idxr[p], sem_i[p]).wait()
        pltpu.make_async_copy(
            col_hbm.at[pl.ds(ebase, CHUNK)], idxc[p], sem_i[p]).wait()

    def fire_gather(p):
        pltpu.async_copy(z_hbm.at[idxr[p]], zr[p], sem_g[p])
        pltpu.async_copy(z_hbm.at[idxc[p]], zc[p], sem_g[p])

    def wait_gather(p):
        pltpu.make_async_copy(z_hbm.at[idxr[p]], zr[p], sem_g[p]).wait()
        pltpu.make_async_copy(z_hbm.at[idxc[p]], zc[p], sem_g[p]).wait()

    def fire_out(c, p):
        ebase = base + c * CHUNK
        pltpu.async_copy(outv[p], out_hbm.at[pl.ds(ebase, CHUNK)], sem_o[p])

    def wait_out(c, p):
        ebase = base + c * CHUNK
        pltpu.make_async_copy(
            outv[p], out_hbm.at[pl.ds(ebase, CHUNK)], sem_o[p]).wait()

    def step(c, p):
        q = 1 - p
        wait_gather(p)

        @pl.when(c + 2 < N_CHUNKS)
        def _():
            fire_idx(c + 2, p)

        @pl.when(c + 1 < N_CHUNKS)
        def _():
            wait_idx(c + 1, q)
            fire_gather(q)

        @pl.when(c >= 2)
        def _():
            wait_out(c - 2, p)

        _dot_chunk(zr[p], zc[p], outv[p])
        fire_out(c, p)

    fire_idx(0, 0)
    fire_idx(1, 1)
    wait_idx(0, 0)
    fire_gather(0)

    def pair(k, carry):
        c = k * 2
        step(c, 0)
        step(c + 1, 1)
        return carry

    lax.fori_loop(0, N_CHUNKS // 2, pair, 0)
    step(N_CHUNKS - 1, 0)

    wait_out(N_CHUNKS - 2, 1)
    wait_out(N_CHUNKS - 1, 0)


@jax.jit
def _decode(z, row, col):
    mesh = plsc.VectorSubcoreMesh(core_axis_name="c", subcore_axis_name="s")
    f = pl.kernel(
        _sc_body,
        mesh=mesh,
        compiler_params=pltpu.CompilerParams(
            use_tc_tiling_on_sc=False, needs_layout_passes=False
        ),
        out_type=jax.ShapeDtypeStruct((N_EDGES,), jnp.float32),
        scratch_types=[
            pltpu.VMEM((CHUNK,), jnp.int32),
            pltpu.VMEM((CHUNK,), jnp.int32),
            pltpu.VMEM((CHUNK,), jnp.int32),
            pltpu.VMEM((CHUNK,), jnp.int32),
            pltpu.VMEM((CHUNK, DIM), jnp.float32),
            pltpu.VMEM((CHUNK, DIM), jnp.float32),
            pltpu.VMEM((CHUNK, DIM), jnp.float32),
            pltpu.VMEM((CHUNK, DIM), jnp.float32),
            pltpu.VMEM((CHUNK,), jnp.float32),
            pltpu.VMEM((CHUNK,), jnp.float32),
            pltpu.SemaphoreType.DMA,
            pltpu.SemaphoreType.DMA,
            pltpu.SemaphoreType.DMA,
            pltpu.SemaphoreType.DMA,
            pltpu.SemaphoreType.DMA,
            pltpu.SemaphoreType.DMA,
        ],
    )
    return f(z, row, col)


def kernel(z, edge_index):
    row = edge_index[0].astype(jnp.int32)
    col = edge_index[1].astype(jnp.int32)
    return _decode(z, row, col)


# speedup vs baseline: 5.8786x; 5.8786x over previous
import jax
import jax.numpy as jnp
from jax import lax
from jax.experimental import pallas as pl
from jax.experimental.pallas import tpu as pltpu
from jax.experimental.pallas import tpu_sc as plsc

N_NODES = 10000
DIM = 128
N_EDGES = 320000

NC = 2
NS = 16
NW = NC * NS

EDGES_PER_W = N_EDGES // NW
CHUNK = 80
N_CHUNKS = EDGES_PER_W // CHUNK
BLK = 16


def _dot_chunk(zr_ref, zc_ref, pbuf, out_ref):
    nblk = CHUNK // BLK
    nseg = DIM // 16
    col_ids = [lax.iota(jnp.int32, 16) * BLK + l for l in range(BLK)]

    def block(b, carry):
        base_e = b * BLK
        for e in range(BLK):
            row = base_e + e
            prods = [
                zr_ref[row, pl.ds(l * 16, 16)] * zc_ref[row, pl.ds(l * 16, 16)]
                for l in range(nseg)
            ]
            while len(prods) > 1:
                prods = [
                    prods[i] + prods[i + 1] for i in range(0, len(prods), 2)
                ]
            pbuf[pl.ds(e * BLK, BLK)] = prods[0]
        acc = jnp.zeros((16,), jnp.float32)
        for l in range(BLK):
            acc = acc + plsc.load_gather(pbuf, [col_ids[l]])
        out_ref[pl.ds(b * BLK, BLK)] = 1.0 / (1.0 + jnp.exp(-acc))
        return carry

    lax.fori_loop(0, nblk, block, 0)


def _sc_body(z_hbm, row_hbm, col_hbm, out_hbm,
             idxr0, idxc0, idxr1, idxc1,
             zr0, zc0, zr1, zc1, outv0, outv1, pbuf,
             sg0, sg1, si0, si1, so0, so1):
    idxr = (idxr0, idxr1)
    idxc = (idxc0, idxc1)
    zr = (zr0, zr1)
    zc = (zc0, zc1)
    outv = (outv0, outv1)
    sem_g = (sg0, sg1)
    sem_i = (si0, si1)
    sem_o = (so0, so1)

    wid = lax.axis_index("c") * NS + lax.axis_index("s")
    base = wid * EDGES_PER_W

    def fire_idx(c, p):
        ebase = base + c * CHUNK
        pltpu.async_copy(row_hbm.at[pl.ds(ebase, CHUNK)], idxr[p], sem_i[p])
        pltpu.async_copy(col_hbm.at[pl.ds(ebase, CHUNK)], idxc[p], sem_i[p])

    def wait_idx(c, p):
        ebase = base + c * CHUNK
        pltpu.make_async_copy(
            row_hbm.at[pl.ds(ebase, CHUNK)], idxr[p], sem_i[p]).wait()
        pltpu.make_async_copy(
            col_hbm.at[pl.ds(ebase, CHUNK)], idxc[p], sem_i[p]).wait()

    def fire_gather(p):
        pltpu.async_copy(z_hbm.at[idxr[p]], zr[p], sem_g[p])
        pltpu.async_copy(z_hbm.at[idxc[p]], zc[p], sem_g[p])

    def wait_gather(p):
        pltpu.make_async_copy(z_hbm.at[idxr[p]], zr[p], sem_g[p]).wait()
        pltpu.make_async_copy(z_hbm.at[idxc[p]], zc[p], sem_g[p]).wait()

    def fire_out(c, p):
        ebase = base + c * CHUNK
        pltpu.async_copy(outv[p], out_hbm.at[pl.ds(ebase, CHUNK)], sem_o[p])

    def wait_out(c, p):
        ebase = base + c * CHUNK
        pltpu.make_async_copy(
            outv[p], out_hbm.at[pl.ds(ebase, CHUNK)], sem_o[p]).wait()

    def step(c, p):
        q = 1 - p
        wait_gather(p)

        @pl.when(c + 2 < N_CHUNKS)
        def _():
            fire_idx(c + 2, p)

        @pl.when(c + 1 < N_CHUNKS)
        def _():
            wait_idx(c + 1, q)
            fire_gather(q)

        @pl.when(c >= 2)
        def _():
            wait_out(c - 2, p)

        _dot_chunk(zr[p], zc[p], pbuf, outv[p])
        fire_out(c, p)

    fire_idx(0, 0)
    fire_idx(1, 1)
    wait_idx(0, 0)
    fire_gather(0)

    def pair(k, carry):
        c = k * 2
        step(c, 0)
        step(c + 1, 1)
        return carry

    lax.fori_loop(0, N_CHUNKS // 2, pair, 0)
    step(N_CHUNKS - 1, 0)

    wait_out(N_CHUNKS - 2, 1)
    wait_out(N_CHUNKS - 1, 0)


@jax.jit
def _decode(z, row, col):
    mesh = plsc.VectorSubcoreMesh(core_axis_name="c", subcore_axis_name="s")
    f = pl.kernel(
        _sc_body,
        mesh=mesh,
        compiler_params=pltpu.CompilerParams(
            use_tc_tiling_on_sc=False, needs_layout_passes=False
        ),
        out_type=jax.ShapeDtypeStruct((N_EDGES,), jnp.float32),
        scratch_types=[
            pltpu.VMEM((CHUNK,), jnp.int32),
            pltpu.VMEM((CHUNK,), jnp.int32),
            pltpu.VMEM((CHUNK,), jnp.int32),
            pltpu.VMEM((CHUNK,), jnp.int32),
            pltpu.VMEM((CHUNK, DIM), jnp.float32),
            pltpu.VMEM((CHUNK, DIM), jnp.float32),
            pltpu.VMEM((CHUNK, DIM), jnp.float32),
            pltpu.VMEM((CHUNK, DIM), jnp.float32),
            pltpu.VMEM((CHUNK,), jnp.float32),
            pltpu.VMEM((CHUNK,), jnp.float32),
            pltpu.VMEM((BLK * BLK,), jnp.float32),
            pltpu.SemaphoreType.DMA,
            pltpu.SemaphoreType.DMA,
            pltpu.SemaphoreType.DMA,
            pltpu.SemaphoreType.DMA,
            pltpu.SemaphoreType.DMA,
            pltpu.SemaphoreType.DMA,
        ],
    )
    return f(z, row, col)


def kernel(z, edge_index):
    row = edge_index[0].astype(jnp.int32)
    col = edge_index[1].astype(jnp.int32)
    return _decode(z, row, col)
